# baseline (speedup 1.0000x reference)
import functools

import jax
import jax.numpy as jnp
from jax.experimental import pallas as pl
from jax.experimental.pallas import tpu as pltpu

_K_I = 2


def _router_kernel(x_ref, gw_ref, b_ref, wt_ref):
    x = x_ref[...]
    gw = gw_ref[...]
    logits = jax.lax.dot_general(
        x, gw, (((1,), (1,)), ((), ())), preferred_element_type=jnp.float32)
    logits = logits + b_ref[...]
    t, e = logits.shape
    e_iota = jax.lax.broadcasted_iota(jnp.int32, (t, e), 1)
    m1 = jnp.max(logits, axis=1, keepdims=True)
    i1 = jnp.min(jnp.where(logits == m1, e_iota, e), axis=1, keepdims=True)
    masked = jnp.where(e_iota == i1, -jnp.inf, logits)
    m2 = jnp.max(masked, axis=1, keepdims=True)
    i2 = jnp.min(jnp.where(masked == m2, e_iota, e), axis=1, keepdims=True)
    w1 = 1.0 / (1.0 + jnp.exp(m2 - m1))
    w2 = 1.0 - w1
    wt_ref[...] = (jnp.where(e_iota == i1, w1, 0.0)
                   + jnp.where(e_iota == i2, w2, 0.0))


def _ffn_kernel(es_ref, na_ref, x_ref, wt_ref,
                ga_ref, gb_ref, ua_ref, ub_ref, da_ref, db_ref, o_ref):
    del es_ref, na_ref
    s = pl.program_id(0)
    i = pl.program_id(1)
    x = x_ref[...]
    bh = x.shape[1] // 2
    xa = x[:, :bh]
    xb = x[:, bh:]
    g = (jnp.dot(xa, ga_ref[0], preferred_element_type=jnp.float32)
         + jnp.dot(xb, gb_ref[0], preferred_element_type=jnp.float32))
    u = (jnp.dot(xa, ua_ref[0], preferred_element_type=jnp.float32)
         + jnp.dot(xb, ub_ref[0], preferred_element_type=jnp.float32))
    act = g * jax.nn.sigmoid(g) * u
    bi2 = act.shape[1] // 2
    part = (jnp.dot(act[:, :bi2], da_ref[0], preferred_element_type=jnp.float32)
            + jnp.dot(act[:, bi2:], db_ref[0], preferred_element_type=jnp.float32))
    wt = wt_ref[...]
    col = jax.lax.broadcasted_iota(jnp.int32, wt.shape, 1)
    w = jnp.sum(jnp.where(col == s, wt, 0.0), axis=1, keepdims=True)
    contrib = part * w
    first = jnp.logical_and(s == 0, i == 0)

    @pl.when(first)
    def _():
        o_ref[...] = contrib

    @pl.when(jnp.logical_not(first))
    def _():
        o_ref[...] += contrib


@functools.partial(jax.jit, static_argnames=())
def kernel(x, gate_w, expert_bias, gate_proj, up_proj, down_proj):
    b, s_len, h = x.shape
    e = gate_proj.shape[0]
    inner = gate_proj.shape[2]
    t = b * s_len
    x2 = x.reshape(t, h)

    wt = pl.pallas_call(
        _router_kernel,
        out_shape=jax.ShapeDtypeStruct((t, e), jnp.float32),
    )(x2, gate_w, expert_bias.reshape(1, e))

    active = jnp.any(wt > 0.0, axis=0)
    num_active = jnp.sum(active.astype(jnp.int32))
    order = jnp.argsort(jnp.logical_not(active), stable=True).astype(jnp.int32)
    last = order[num_active - 1]
    steps = jnp.arange(e, dtype=jnp.int32)
    es = jnp.where(steps < num_active, order, last)
    wt_sched = jnp.where(steps[None, :] < num_active, wt[:, es], 0.0)
    na = num_active.reshape(1)

    k_i = _K_I
    bi = inner // k_i
    bh = h // 2
    bi2 = bi // 2

    def gu_a(s, i, es, na):
        return (es[s], 0, jnp.where(s < na[0], i, k_i - 1))

    def gu_b(s, i, es, na):
        return (es[s], 1, jnp.where(s < na[0], i, k_i - 1))

    def d_a(s, i, es, na):
        return (es[s], 2 * jnp.where(s < na[0], i, k_i - 1), 0)

    def d_b(s, i, es, na):
        return (es[s], 2 * jnp.where(s < na[0], i, k_i - 1) + 1, 0)

    out = pl.pallas_call(
        _ffn_kernel,
        grid_spec=pltpu.PrefetchScalarGridSpec(
            num_scalar_prefetch=2,
            grid=(e, k_i),
            in_specs=[
                pl.BlockSpec((t, h), lambda s, i, es, na: (0, 0)),
                pl.BlockSpec((t, e), lambda s, i, es, na: (0, 0)),
                pl.BlockSpec((1, bh, bi), gu_a),
                pl.BlockSpec((1, bh, bi), gu_b),
                pl.BlockSpec((1, bh, bi), gu_a),
                pl.BlockSpec((1, bh, bi), gu_b),
                pl.BlockSpec((1, bi2, h), d_a),
                pl.BlockSpec((1, bi2, h), d_b),
            ],
            out_specs=pl.BlockSpec((t, h), lambda s, i, es, na: (0, 0)),
        ),
        out_shape=jax.ShapeDtypeStruct((t, h), jnp.float32),
    )(es, na, x2, wt_sched, gate_proj, gate_proj, up_proj, up_proj,
      down_proj, down_proj)

    return out.reshape(b, s_len, h)


# speedup vs baseline: 1.1064x; 1.1064x over previous
import functools

import jax
import jax.numpy as jnp
from jax.experimental import pallas as pl
from jax.experimental.pallas import tpu as pltpu


def _router_kernel(x_ref, gw_ref, b_ref, wt_ref):
    x = x_ref[...]
    gw = gw_ref[...]
    logits = jax.lax.dot_general(
        x, gw, (((1,), (1,)), ((), ())), preferred_element_type=jnp.float32)
    logits = logits + b_ref[...]
    t, e = logits.shape
    e_iota = jax.lax.broadcasted_iota(jnp.int32, (t, e), 1)
    m1 = jnp.max(logits, axis=1, keepdims=True)
    i1 = jnp.min(jnp.where(logits == m1, e_iota, e), axis=1, keepdims=True)
    masked = jnp.where(e_iota == i1, -jnp.inf, logits)
    m2 = jnp.max(masked, axis=1, keepdims=True)
    i2 = jnp.min(jnp.where(masked == m2, e_iota, e), axis=1, keepdims=True)
    w1 = 1.0 / (1.0 + jnp.exp(m2 - m1))
    w2 = 1.0 - w1
    wt_ref[...] = (jnp.where(e_iota == i1, w1, 0.0)
                   + jnp.where(e_iota == i2, w2, 0.0))


def _ffn_kernel(es_ref, na_ref, x_ref, wt_ref, g_ref, u_ref, d_ref, o_ref):
    del es_ref
    s = pl.program_id(0)
    i = pl.program_id(1)

    @pl.when(s < na_ref[0])
    def _():
        x = x_ref[...]
        g = jnp.dot(x, g_ref[0], preferred_element_type=jnp.float32)
        u = jnp.dot(x, u_ref[0], preferred_element_type=jnp.float32)
        act = g * jax.nn.sigmoid(g) * u
        part = jnp.dot(act, d_ref[0], preferred_element_type=jnp.float32)
        wt = wt_ref[...]
        col = jax.lax.broadcasted_iota(jnp.int32, wt.shape, 1)
        w = jnp.sum(jnp.where(col == s, wt, 0.0), axis=1, keepdims=True)
        contrib = part * w
        first = jnp.logical_and(s == 0, i == 0)

        @pl.when(first)
        def _():
            o_ref[...] = contrib

        @pl.when(jnp.logical_not(first))
        def _():
            o_ref[...] += contrib


@functools.partial(jax.jit, static_argnames=())
def kernel(x, gate_w, expert_bias, gate_proj, up_proj, down_proj):
    b, s_len, h = x.shape
    e = gate_proj.shape[0]
    inner = gate_proj.shape[2]
    t = b * s_len
    x2 = x.reshape(t, h)

    wt = pl.pallas_call(
        _router_kernel,
        out_shape=jax.ShapeDtypeStruct((t, e), jnp.float32),
    )(x2, gate_w, expert_bias.reshape(1, e))

    active = jnp.any(wt > 0.0, axis=0)
    num_active = jnp.sum(active.astype(jnp.int32))
    order = jnp.argsort(jnp.logical_not(active), stable=True).astype(jnp.int32)
    last = order[num_active - 1]
    steps = jnp.arange(e, dtype=jnp.int32)
    es = jnp.where(steps < num_active, order, last)
    wt_sched = jnp.where(steps[None, :] < num_active, wt[:, es], 0.0)
    na = num_active.reshape(1)

    k_i = 2
    bi = inner // k_i

    def w_idx(s, i, es, na):
        fi = jnp.where(s < na[0], i, k_i - 1)
        return (es[s], 0, fi)

    def d_idx(s, i, es, na):
        fi = jnp.where(s < na[0], i, k_i - 1)
        return (es[s], fi, 0)

    out = pl.pallas_call(
        _ffn_kernel,
        grid_spec=pltpu.PrefetchScalarGridSpec(
            num_scalar_prefetch=2,
            grid=(e, k_i),
            in_specs=[
                pl.BlockSpec((t, h), lambda s, i, es, na: (0, 0)),
                pl.BlockSpec((t, e), lambda s, i, es, na: (0, 0)),
                pl.BlockSpec((1, h, bi), w_idx),
                pl.BlockSpec((1, h, bi), w_idx),
                pl.BlockSpec((1, bi, h), d_idx),
            ],
            out_specs=pl.BlockSpec((t, h), lambda s, i, es, na: (0, 0)),
        ),
        out_shape=jax.ShapeDtypeStruct((t, h), jnp.float32),
    )(es, na, x2, wt_sched, gate_proj, up_proj, down_proj)

    return out.reshape(b, s_len, h)


# speedup vs baseline: 1.1455x; 1.0354x over previous
import functools

import jax
import jax.numpy as jnp
from jax.experimental import pallas as pl
from jax.experimental.pallas import tpu as pltpu


def _router_kernel(x_ref, gw_ref, b_ref, wts_ref, es_ref, na_ref):
    x = x_ref[...]
    gw = gw_ref[...]
    logits = jax.lax.dot_general(
        x, gw, (((1,), (1,)), ((), ())), preferred_element_type=jnp.float32)
    logits = logits + b_ref[...]
    t, e = logits.shape
    e_iota = jax.lax.broadcasted_iota(jnp.int32, (t, e), 1)
    m1 = jnp.max(logits, axis=1, keepdims=True)
    i1 = jnp.min(jnp.where(logits == m1, e_iota, e), axis=1, keepdims=True)
    masked = jnp.where(e_iota == i1, -jnp.inf, logits)
    m2 = jnp.max(masked, axis=1, keepdims=True)
    i2 = jnp.min(jnp.where(masked == m2, e_iota, e), axis=1, keepdims=True)
    w1 = 1.0 / (1.0 + jnp.exp(m2 - m1))
    w2 = 1.0 - w1
    wt = (jnp.where(e_iota == i1, w1, 0.0)
          + jnp.where(e_iota == i2, w2, 0.0))

    wt_t = wt.T
    active = (jnp.max(wt_t, axis=1, keepdims=True) > 0.0).astype(jnp.float32)
    na = jnp.sum(active)
    r_iota = jax.lax.broadcasted_iota(jnp.int32, (e, e), 0).astype(jnp.float32)
    c_iota = jax.lax.broadcasted_iota(jnp.int32, (e, e), 1).astype(jnp.float32)
    lt = (c_iota < r_iota).astype(jnp.float32)
    rank = jax.lax.dot_general(
        lt, active, (((1,), (0,)), ((), ())),
        preferred_element_type=jnp.float32)
    sel = ((rank == c_iota) * active)
    ids = jax.lax.broadcasted_iota(jnp.int32, (1, e), 1).astype(jnp.float32)
    es_row = jax.lax.dot_general(
        ids, sel, (((1,), (0,)), ((), ())),
        preferred_element_type=jnp.float32)
    last = jnp.sum(jnp.where(ids == na - 1.0, es_row, 0.0))
    es_pad = jnp.where(ids < na, es_row, last)
    es_ref[...] = es_pad.astype(jnp.int32)
    na_ref[...] = jnp.full((1, 1), na, jnp.float32).astype(jnp.int32)
    wts_ref[...] = jnp.dot(wt, sel, preferred_element_type=jnp.float32)


def _ffn_kernel(es_ref, na_ref, x_ref, wt_ref, g_ref, u_ref, d_ref, o_ref):
    del es_ref
    s = pl.program_id(0)
    i = pl.program_id(1)

    @pl.when(s < na_ref[0])
    def _():
        x = x_ref[...]
        g = jnp.dot(x, g_ref[0], preferred_element_type=jnp.float32)
        u = jnp.dot(x, u_ref[0], preferred_element_type=jnp.float32)
        act = g * jax.nn.sigmoid(g) * u
        part = jnp.dot(act, d_ref[0], preferred_element_type=jnp.float32)
        wt = wt_ref[...]
        col = jax.lax.broadcasted_iota(jnp.int32, wt.shape, 1)
        w = jnp.sum(jnp.where(col == s, wt, 0.0), axis=1, keepdims=True)
        contrib = part * w
        first = jnp.logical_and(s == 0, i == 0)

        @pl.when(first)
        def _():
            o_ref[...] = contrib

        @pl.when(jnp.logical_not(first))
        def _():
            o_ref[...] += contrib


@functools.partial(jax.jit, static_argnames=())
def kernel(x, gate_w, expert_bias, gate_proj, up_proj, down_proj):
    b, s_len, h = x.shape
    e = gate_proj.shape[0]
    inner = gate_proj.shape[2]
    t = b * s_len
    x2 = x.reshape(t, h)

    wt_sched, es2, na2 = pl.pallas_call(
        _router_kernel,
        out_shape=[
            jax.ShapeDtypeStruct((t, e), jnp.float32),
            jax.ShapeDtypeStruct((1, e), jnp.int32),
            jax.ShapeDtypeStruct((1, 1), jnp.int32),
        ],
    )(x2, gate_w, expert_bias.reshape(1, e))
    es = es2.reshape(e)
    na = na2.reshape(1)

    k_i = 2
    bi = inner // k_i

    def w_idx(s, i, es, na):
        fi = jnp.where(s < na[0], i, k_i - 1)
        return (es[s], 0, fi)

    def d_idx(s, i, es, na):
        fi = jnp.where(s < na[0], i, k_i - 1)
        return (es[s], fi, 0)

    out = pl.pallas_call(
        _ffn_kernel,
        grid_spec=pltpu.PrefetchScalarGridSpec(
            num_scalar_prefetch=2,
            grid=(e, k_i),
            in_specs=[
                pl.BlockSpec((t, h), lambda s, i, es, na: (0, 0)),
                pl.BlockSpec((t, e), lambda s, i, es, na: (0, 0)),
                pl.BlockSpec((1, h, bi), w_idx),
                pl.BlockSpec((1, h, bi), w_idx),
                pl.BlockSpec((1, bi, h), d_idx),
            ],
            out_specs=pl.BlockSpec((t, h), lambda s, i, es, na: (0, 0)),
        ),
        out_shape=jax.ShapeDtypeStruct((t, h), jnp.float32),
    )(es, na, x2, wt_sched, gate_proj, up_proj, down_proj)

    return out.reshape(b, s_len, h)
